# baseline (speedup 1.0000x reference)
import functools
import numpy as np
import jax
import jax.numpy as jnp
from jax.experimental import pallas as pl
from jax.experimental.pallas import tpu as pltpu

_D_MODEL = 768
_N_HEADS = 12
_D_FF = 3072
_E_LAYERS = 3
_FACTOR = 5
_LN_EPS = 1e-5
_BN_EPS = 1e-5
_DH = _D_MODEL // _N_HEADS
_UPAD = 64
_NEG_INF = float('-inf')


def _tf2x32_raw(k1, k2, x0, x1):
    def rnd(v0, v1, r):
        v0 = (v0 + v1).astype(np.uint32)
        v1 = ((v1 << np.uint32(r)) | (v1 >> np.uint32(32 - r))).astype(np.uint32)
        return v0, v1 ^ v0
    rot0 = (13, 15, 26, 6)
    rot1 = (17, 29, 16, 24)
    ks0 = np.uint32(k1)
    ks1 = np.uint32(k2)
    ks2 = np.uint32(ks0 ^ ks1 ^ np.uint32(0x1BD11BDA))
    x0 = (x0 + ks0).astype(np.uint32)
    x1 = (x1 + ks1).astype(np.uint32)
    sched = [(rot0, ks1, ks2, 1), (rot1, ks2, ks0, 2), (rot0, ks0, ks1, 3),
             (rot1, ks1, ks2, 4), (rot0, ks2, ks0, 5)]
    for rots, a0, a1, c in sched:
        for r in rots:
            x0, x1 = rnd(x0, x1, r)
        x0 = (x0 + a0).astype(np.uint32)
        x1 = (x1 + a1 + np.uint32(c)).astype(np.uint32)
    return x0, x1


def _tf2x32(key, count):
    flat = count.ravel().astype(np.uint32)
    odd = flat.shape[0] % 2
    if odd:
        flat = np.concatenate([flat, np.uint32([0])])
    half = flat.shape[0] // 2
    o0, o1 = _tf2x32_raw(key[0], key[1], flat[:half], flat[half:])
    out = np.concatenate([o0, o1])
    return (out[:-1] if odd else out).reshape(count.shape)


def _np_seed(x):
    return np.array([(x >> 32) & 0xFFFFFFFF, x & 0xFFFFFFFF], np.uint32)


def _np_bits32(key, n, part):
    if part:
        b1, b2 = _tf2x32_raw(key[0], key[1], np.zeros(n, np.uint32),
                             np.arange(n, dtype=np.uint32))
        return b1 ^ b2
    return _tf2x32(key, np.arange(n, dtype=np.uint32))


def _np_split2(key, part):
    if part:
        b1, b2 = _tf2x32_raw(key[0], key[1], np.zeros(2, np.uint32),
                             np.arange(2, dtype=np.uint32))
        return np.stack([b1, b2], axis=1)
    return _tf2x32(key, np.arange(4, dtype=np.uint32)).reshape(2, 2)


def _np_randint(key, shape, minval, maxval, part):
    n = int(np.prod(shape))
    k1, k2 = _np_split2(key, part)
    hi = _np_bits32(k1, n, part)
    lo = _np_bits32(k2, n, part)
    span = np.uint32(maxval - minval)
    mult = np.uint32((2 ** 16) % int(span))
    mult = np.uint32((int(mult) * int(mult)) % int(span))
    off = ((hi % span) * mult + (lo % span)) % span
    return (np.int32(minval) + off.astype(np.int32)).reshape(shape)


_CONST_CACHE = {}


def _u_of(L):
    return min(int(_FACTOR * np.ceil(np.log(L))), L)


def _sample_consts(layer_idx, L):
    key = ('c', layer_idx, L)
    if key not in _CONST_CACHE:
        part = bool(jax.config.jax_threefry_partitionable)
        rng_key = _tf2x32(_np_seed(42), _np_seed(layer_idx))
        idx = _np_randint(rng_key, (L, _u_of(L)), 0, L, part)
        cnt = np.zeros((L, L), dtype=np.float32)
        np.add.at(cnt, (np.arange(L)[:, None], idx), 1.0)
        amask = np.where(cnt > 0.0, 0.0, _NEG_INF).astype(np.float32)
        _CONST_CACHE[key] = (cnt.T.copy(), amask.T.copy())
    return _CONST_CACHE[key]


def _pe_table(L, d_model):
    key = ('pe', L, d_model)
    if key not in _CONST_CACHE:
        pe = np.zeros((L, d_model), dtype=np.float32)
        pos = np.arange(L, dtype=np.float32)[:, None]
        div = np.exp(np.arange(0, d_model, 2, dtype=np.float32)
                     * -(np.log(10000.0) / d_model))
        pe[:, 0::2] = np.sin(pos * div)
        pe[:, 1::2] = np.cos(pos * div)
        _CONST_CACHE[key] = pe
    return _CONST_CACHE[key]


def _dot(a, b, ca, cb):
    return jax.lax.dot_general(
        a, b, (((ca,), (cb,)), ((), ())),
        preferred_element_type=jnp.float32)


def _ln(x, g, b):
    m = jnp.mean(x, axis=-1, keepdims=True)
    xc = x - m
    v = jnp.mean(xc * xc, axis=-1, keepdims=True)
    return xc / jnp.sqrt(v + _LN_EPS) * g + b


def _shift_down(x):
    return jnp.concatenate([x[-1:], x[:-1]], axis=0)


def _shift_up(x):
    return jnp.concatenate([x[1:], x[:1]], axis=0)


def _embed_body(x_ref, wt_ref, pe_ref, o_ref):
    x = x_ref[...]
    y = (_dot(_shift_down(x), wt_ref[0], 1, 0)
         + _dot(x, wt_ref[1], 1, 0)
         + _dot(_shift_up(x), wt_ref[2], 1, 0))
    o_ref[...] = y + pe_ref[...]


def _qkv_body(h_ref, wq_ref, wk_ref, wv_ref, bq_ref, bk_ref, bv_ref,
              q_ref, k_ref, v_ref):
    h = h_ref[...]
    q_ref[...] = _dot(h, wq_ref[...], 1, 0) + bq_ref[...]
    k_ref[...] = _dot(h, wk_ref[...], 1, 0) + bk_ref[...]
    v_ref[...] = _dot(h, wv_ref[...], 1, 0) + bv_ref[...]


def _mtopk_body(q_ref, k_ref, cnt_ref, msk_ref, o_ref, mx_scr, sm_scr,
                *, L, u, n_chunks):
    c = pl.program_id(0)
    msk = msk_ref[...]
    cnt = cnt_ref[...]
    for h in range(_N_HEADS):
        sl = slice(h * _DH, (h + 1) * _DH)
        st = _dot(k_ref[:, sl], q_ref[:, sl], 1, 1)
        cmax = jnp.max(st + msk, axis=0, keepdims=True)
        csum = jnp.sum(st * cnt, axis=0, keepdims=True)

        @pl.when(c == 0)
        def _init():
            mx_scr[h:h + 1, :] = cmax
            sm_scr[h:h + 1, :] = csum

        @pl.when(c > 0)
        def _acc():
            mx_scr[h:h + 1, :] = jnp.maximum(mx_scr[h:h + 1, :], cmax)
            sm_scr[h:h + 1, :] = sm_scr[h:h + 1, :] + csum

    @pl.when(c == n_chunks - 1)
    def _finish():
        mv = mx_scr[0:_N_HEADS, :] - sm_scr[0:_N_HEADS, :] * (1.0 / L)
        _topk_write(mv, o_ref, L, u)


def _topk_write(mv, o_ref, L, u):
    iota = jax.lax.broadcasted_iota(jnp.int32, (_N_HEADS, L), 1)
    cols = []
    for _ in range(u):
        cur = jnp.max(mv, axis=1, keepdims=True)
        idx_t = jnp.min(jnp.where(mv == cur, iota, L), axis=1, keepdims=True)
        cols.append(idx_t)
        mv = jnp.where(iota == idx_t, _NEG_INF, mv)
    if u < _UPAD:
        cols.append(jnp.full((_N_HEADS, _UPAD - u), L, jnp.int32))
    inds = jnp.concatenate(cols, axis=1)
    o_ref[...] = jnp.concatenate(
        [inds, jnp.zeros((16 - _N_HEADS, _UPAD), jnp.int32)], axis=0)


def _attn_body(q_ref, k_ref, v_ref, i_ref, o_ref, *, L):
    out_cols = []
    row_iota = jax.lax.broadcasted_iota(jnp.int32, (L, _UPAD), 0)
    for h in range(_N_HEADS):
        sl = slice(h * _DH, (h + 1) * _DH)
        q_h = q_ref[:, sl]
        k_h = k_ref[:, sl]
        v_h = v_ref[:, sl]
        idx = i_ref[h:h + 1, :]
        oht = (row_iota == idx).astype(jnp.float32)
        q_red = _dot(oht, q_h, 0, 0)
        scores = _dot(q_red, k_h, 1, 1) * (1.0 / np.sqrt(_DH))
        smax = jnp.max(scores, axis=1, keepdims=True)
        e = jnp.exp(scores - smax)
        p = e / jnp.sum(e, axis=1, keepdims=True)
        upd = _dot(p, v_h, 1, 0)
        ctx = _dot(oht, upd, 1, 0)
        sel = jnp.sum(oht, axis=1, keepdims=True)
        vm = jnp.sum(v_h, axis=0, keepdims=True) * (1.0 / L)
        out_cols.append(ctx + (1.0 - sel) * vm)
    o_ref[...] = jnp.concatenate(out_cols, axis=1)


def _post_body(h_ref, ctx_ref, wo_ref, bo_ref, g1_ref, b1_ref,
               w1_ref, c1_ref, w2_ref, c2_ref, g2_ref, b2_ref, *out_refs):
    x = h_ref[...] + _dot(ctx_ref[...], wo_ref[...], 1, 0) + bo_ref[...]
    x = _ln(x, g1_ref[...], b1_ref[...])
    y = jnp.maximum(_dot(x, w1_ref[...], 1, 0) + c1_ref[...], 0.0)
    y = _dot(y, w2_ref[...], 1, 0) + c2_ref[...]
    out = _ln(x + y, g2_ref[...], b2_ref[...])
    if len(out_refs) == 1:
        out_refs[0][...] = out
    else:
        BL = out.shape[0]
        r2 = 2 * jax.lax.broadcasted_iota(jnp.int32, (BL // 2, BL), 0)
        cj = jax.lax.broadcasted_iota(jnp.int32, (BL // 2, BL), 1)
        out_refs[0][...] = _dot((cj == r2).astype(jnp.float32), out, 1, 0)
        out_refs[1][...] = _dot((cj == r2 + 1).astype(jnp.float32), out, 1, 0)


def _distil_body(he_ref, ho_ref, wt_ref, b_ref, bnm_ref, bnv_ref,
                 bng_ref, bnb_ref, o_ref):
    he = he_ref[...]
    ho = ho_ref[...]

    def bn_elu(z):
        z = z + b_ref[...]
        z = (z - bnm_ref[...]) / jnp.sqrt(bnv_ref[...] + _BN_EPS) \
            * bng_ref[...] + bnb_ref[...]
        return jnp.where(z > 0.0, z, jnp.exp(z) - 1.0)

    w0, w1, w2 = wt_ref[0], wt_ref[1], wt_ref[2]
    e_even = bn_elu(_dot(_shift_down(ho), w0, 1, 0) + _dot(he, w1, 1, 0)
                    + _dot(ho, w2, 1, 0))
    e_odd = bn_elu(_dot(he, w0, 1, 0) + _dot(ho, w1, 1, 0)
                   + _dot(_shift_up(he), w2, 1, 0))
    prev_odd = jnp.concatenate(
        [jnp.full_like(e_odd[:1], _NEG_INF), e_odd[:-1]], axis=0)
    o_ref[...] = jnp.maximum(jnp.maximum(prev_odd, e_even), e_odd)


def _final_body(h_ref, g_ref, b_ref, o_ref, *, L):
    x = _ln(h_ref[...], g_ref[...], b_ref[...])
    o_ref[...] = jnp.sum(x, axis=0, keepdims=True) * (1.0 / L)


def _full(a):
    return pl.BlockSpec(a.shape, lambda: tuple(0 for _ in a.shape))


def _single(body, args, out_shape):
    return pl.pallas_call(
        body,
        in_specs=[_full(a) for a in args],
        out_specs=pl.BlockSpec(out_shape.shape,
                               lambda: tuple(0 for _ in out_shape.shape)),
        out_shape=out_shape,
    )(*args)


def _embed_call(x, wt, pe):
    L = x.shape[0]
    return _single(_embed_body, (x, wt, pe),
                   jax.ShapeDtypeStruct((L, _D_MODEL), jnp.float32))


def _qkv_call(h, p):
    L = h.shape[0]
    BL = 512
    row = lambda n: pl.BlockSpec((1, n), lambda i: (0, 0))
    mat = lambda a: pl.BlockSpec(a.shape, lambda i: (0, 0))
    blk = pl.BlockSpec((BL, _D_MODEL), lambda i: (i, 0))
    return pl.pallas_call(
        _qkv_body,
        grid=(max(L // BL, 1),),
        in_specs=[blk, mat(p['Wq']), mat(p['Wk']), mat(p['Wv']),
                  row(_D_MODEL), row(_D_MODEL), row(_D_MODEL)],
        out_specs=[blk, blk, blk],
        out_shape=[jax.ShapeDtypeStruct((L, _D_MODEL), jnp.float32)] * 3,
    )(h, p['Wq'], p['Wk'], p['Wv'], p['bq'].reshape(1, -1),
      p['bk'].reshape(1, -1), p['bv'].reshape(1, -1))


def _mtopk_call(q, k, cntT, amaskT, u):
    L = q.shape[0]
    chunk = min(512, L)
    n_chunks = L // chunk
    body = functools.partial(_mtopk_body, L=L, u=u, n_chunks=n_chunks)
    return pl.pallas_call(
        body,
        grid=(n_chunks,),
        in_specs=[
            pl.BlockSpec((L, _D_MODEL), lambda c: (0, 0)),
            pl.BlockSpec((chunk, _D_MODEL), lambda c: (c, 0)),
            pl.BlockSpec((chunk, L), lambda c: (c, 0)),
            pl.BlockSpec((chunk, L), lambda c: (c, 0)),
        ],
        out_specs=pl.BlockSpec((16, _UPAD), lambda c: (0, 0)),
        out_shape=jax.ShapeDtypeStruct((16, _UPAD), jnp.int32),
        scratch_shapes=[pltpu.VMEM((16, L), jnp.float32),
                        pltpu.VMEM((16, L), jnp.float32)],
    )(q, k, cntT, amaskT)


def _attn_call(q, k, v, inds):
    L = q.shape[0]
    body = functools.partial(_attn_body, L=L)
    return _single(body, (q, k, v, inds),
                   jax.ShapeDtypeStruct((L, _D_MODEL), jnp.float32))


def _post_call(h, ctx, p, split):
    L = h.shape[0]
    BL = 512
    mat = lambda a: pl.BlockSpec(a.shape, lambda i: (0, 0))
    row = lambda n: pl.BlockSpec((1, n), lambda i: (0, 0))
    blk = pl.BlockSpec((BL, _D_MODEL), lambda i: (i, 0))
    if split:
        half = pl.BlockSpec((BL // 2, _D_MODEL), lambda i: (i, 0))
        out_specs = [half, half]
        out_shape = [jax.ShapeDtypeStruct((L // 2, _D_MODEL), jnp.float32)] * 2
    else:
        out_specs = [blk]
        out_shape = [jax.ShapeDtypeStruct((L, _D_MODEL), jnp.float32)]
    return pl.pallas_call(
        _post_body,
        grid=(L // BL,),
        in_specs=[blk, blk, mat(p['Wo']), row(_D_MODEL), row(_D_MODEL),
                  row(_D_MODEL), mat(p['W1']), row(_D_FF), mat(p['W2']),
                  row(_D_MODEL), row(_D_MODEL), row(_D_MODEL)],
        out_specs=out_specs,
        out_shape=out_shape,
    )(h, ctx, p['Wo'], p['bo'].reshape(1, -1), p['g1'].reshape(1, -1),
      p['b1'].reshape(1, -1), p['W1'], p['c1'].reshape(1, -1), p['W2'],
      p['c2'].reshape(1, -1), p['g2'].reshape(1, -1), p['b2'].reshape(1, -1))


def _distil_call(he, ho, p):
    Lh = he.shape[0]
    wt = jnp.transpose(p['w'], (2, 1, 0))
    args = (he, ho, wt, p['b'].reshape(1, -1),
            p['bn_m'].reshape(1, -1), p['bn_v'].reshape(1, -1),
            p['bn_g'].reshape(1, -1), p['bn_b'].reshape(1, -1))
    return _single(_distil_body, args,
                   jax.ShapeDtypeStruct((Lh, _D_MODEL), jnp.float32))


def _final_call(h, g, b):
    body = functools.partial(_final_body, L=h.shape[0])
    return _single(body, (h, g.reshape(1, -1), b.reshape(1, -1)),
                   jax.ShapeDtypeStruct((1, _D_MODEL), jnp.float32))


def kernel(x_enc, params):
    x = x_enc[0] + 1e-10
    L = x.shape[0]

    wt_tok = jnp.transpose(params['tok_w'], (2, 1, 0))
    pe = jnp.asarray(_pe_table(L, _D_MODEL))
    h = _embed_call(x, wt_tok, pe)

    for i in range(_E_LAYERS):
        L = h.shape[0]
        p = params['layers'][i]
        q, k, v = _qkv_call(h, p)
        cntT, amaskT = _sample_consts(i, L)
        inds = _mtopk_call(q, k, jnp.asarray(cntT), jnp.asarray(amaskT),
                           _u_of(L))
        ctx = _attn_call(q, k, v, inds)
        if i < _E_LAYERS - 1:
            he, ho = _post_call(h, ctx, p, True)
            h = _distil_call(he, ho, params['convs'][i])
        else:
            (h,) = _post_call(h, ctx, p, False)

    return _final_call(h, params['norm_g'], params['norm_b'])


# speedup vs baseline: 1.0295x; 1.0295x over previous
import functools
import numpy as np
import jax
import jax.numpy as jnp
from jax.experimental import pallas as pl

_D_MODEL = 768
_N_HEADS = 12
_D_FF = 3072
_E_LAYERS = 3
_FACTOR = 5
_LN_EPS = 1e-5
_BN_EPS = 1e-5
_DH = _D_MODEL // _N_HEADS
_UPAD = 64
_NEG_INF = float('-inf')


def _tf2x32_raw(k1, k2, x0, x1):
    def rnd(v0, v1, r):
        v0 = (v0 + v1).astype(np.uint32)
        v1 = ((v1 << np.uint32(r)) | (v1 >> np.uint32(32 - r))).astype(np.uint32)
        return v0, v1 ^ v0
    rot0 = (13, 15, 26, 6)
    rot1 = (17, 29, 16, 24)
    ks0 = np.uint32(k1)
    ks1 = np.uint32(k2)
    ks2 = np.uint32(ks0 ^ ks1 ^ np.uint32(0x1BD11BDA))
    x0 = (x0 + ks0).astype(np.uint32)
    x1 = (x1 + ks1).astype(np.uint32)
    sched = [(rot0, ks1, ks2, 1), (rot1, ks2, ks0, 2), (rot0, ks0, ks1, 3),
             (rot1, ks1, ks2, 4), (rot0, ks2, ks0, 5)]
    for rots, a0, a1, c in sched:
        for r in rots:
            x0, x1 = rnd(x0, x1, r)
        x0 = (x0 + a0).astype(np.uint32)
        x1 = (x1 + a1 + np.uint32(c)).astype(np.uint32)
    return x0, x1


def _tf2x32(key, count):
    flat = count.ravel().astype(np.uint32)
    odd = flat.shape[0] % 2
    if odd:
        flat = np.concatenate([flat, np.uint32([0])])
    half = flat.shape[0] // 2
    o0, o1 = _tf2x32_raw(key[0], key[1], flat[:half], flat[half:])
    out = np.concatenate([o0, o1])
    return (out[:-1] if odd else out).reshape(count.shape)


def _np_seed(x):
    return np.array([(x >> 32) & 0xFFFFFFFF, x & 0xFFFFFFFF], np.uint32)


def _np_bits32(key, n, part):
    if part:
        b1, b2 = _tf2x32_raw(key[0], key[1], np.zeros(n, np.uint32),
                             np.arange(n, dtype=np.uint32))
        return b1 ^ b2
    return _tf2x32(key, np.arange(n, dtype=np.uint32))


def _np_split2(key, part):
    if part:
        b1, b2 = _tf2x32_raw(key[0], key[1], np.zeros(2, np.uint32),
                             np.arange(2, dtype=np.uint32))
        return np.stack([b1, b2], axis=1)
    return _tf2x32(key, np.arange(4, dtype=np.uint32)).reshape(2, 2)


def _np_randint(key, shape, minval, maxval, part):
    n = int(np.prod(shape))
    k1, k2 = _np_split2(key, part)
    hi = _np_bits32(k1, n, part)
    lo = _np_bits32(k2, n, part)
    span = np.uint32(maxval - minval)
    mult = np.uint32((2 ** 16) % int(span))
    mult = np.uint32((int(mult) * int(mult)) % int(span))
    off = ((hi % span) * mult + (lo % span)) % span
    return (np.int32(minval) + off.astype(np.int32)).reshape(shape)


_CONST_CACHE = {}


def _u_of(L):
    return min(int(_FACTOR * np.ceil(np.log(L))), L)


def _sample_consts(layer_idx, L):
    key = ('c', layer_idx, L)
    if key not in _CONST_CACHE:
        part = bool(jax.config.jax_threefry_partitionable)
        rng_key = _tf2x32(_np_seed(42), _np_seed(layer_idx))
        idx = _np_randint(rng_key, (L, _u_of(L)), 0, L, part)
        cnt = np.zeros((L, L), dtype=np.float32)
        np.add.at(cnt, (np.arange(L)[:, None], idx), 1.0)
        amask = np.where(cnt > 0.0, 0.0, _NEG_INF).astype(np.float32)
        _CONST_CACHE[key] = (cnt.T.copy(), amask.T.copy())
    return _CONST_CACHE[key]


def _pe_table(L, d_model):
    key = ('pe', L, d_model)
    if key not in _CONST_CACHE:
        pe = np.zeros((L, d_model), dtype=np.float32)
        pos = np.arange(L, dtype=np.float32)[:, None]
        div = np.exp(np.arange(0, d_model, 2, dtype=np.float32)
                     * -(np.log(10000.0) / d_model))
        pe[:, 0::2] = np.sin(pos * div)
        pe[:, 1::2] = np.cos(pos * div)
        _CONST_CACHE[key] = pe
    return _CONST_CACHE[key]


def _dot(a, b, ca, cb):
    return jax.lax.dot_general(
        a, b, (((ca,), (cb,)), ((), ())),
        preferred_element_type=jnp.float32)


def _ln(x, g, b):
    m = jnp.mean(x, axis=-1, keepdims=True)
    xc = x - m
    v = jnp.mean(xc * xc, axis=-1, keepdims=True)
    return xc / jnp.sqrt(v + _LN_EPS) * g + b


def _shift_down(x):
    return jnp.concatenate([x[-1:], x[:-1]], axis=0)


def _shift_up(x):
    return jnp.concatenate([x[1:], x[:1]], axis=0)


def _embed_body(x_ref, wt_ref, pe_ref, o_ref):
    x = x_ref[...]
    y = (_dot(_shift_down(x), wt_ref[0], 1, 0)
         + _dot(x, wt_ref[1], 1, 0)
         + _dot(_shift_up(x), wt_ref[2], 1, 0))
    o_ref[...] = y + pe_ref[...]


def _qkv_body(h_ref, wq_ref, wk_ref, wv_ref, bq_ref, bk_ref, bv_ref,
              q_ref, k_ref, v_ref):
    h = h_ref[...]
    q_ref[...] = _dot(h, wq_ref[...], 1, 0) + bq_ref[...]
    k_ref[...] = _dot(h, wk_ref[...], 1, 0) + bk_ref[...]
    v_ref[...] = _dot(h, wv_ref[...], 1, 0) + bv_ref[...]


def _mtopk_body(q_ref, k_ref, cnt_ref, msk_ref, o_ref, *, L, u):
    chunk = min(512, L)
    n_chunks = L // chunk
    rows = []
    for h in range(_N_HEADS):
        sl = slice(h * _DH, (h + 1) * _DH)
        q_h = q_ref[:, sl]
        run_max = None
        run_sum = None
        for c in range(n_chunks):
            k_c = k_ref[c * chunk:(c + 1) * chunk, sl]
            st = _dot(k_c, q_h, 1, 1)
            masked = st + msk_ref[c * chunk:(c + 1) * chunk, :]
            cmax = jnp.max(masked, axis=0, keepdims=True)
            csum = jnp.sum(st * cnt_ref[c * chunk:(c + 1) * chunk, :],
                           axis=0, keepdims=True)
            run_max = cmax if run_max is None else jnp.maximum(run_max, cmax)
            run_sum = csum if run_sum is None else run_sum + csum
        rows.append(run_max - run_sum * (1.0 / L))
    mv = jnp.concatenate(rows, axis=0)
    _topk_write(mv, o_ref, L, u)


def _topk_write(mv, o_ref, L, u):
    iota = jax.lax.broadcasted_iota(jnp.int32, (_N_HEADS, L), 1)
    cols = []
    for _ in range(u):
        cur = jnp.max(mv, axis=1, keepdims=True)
        idx_t = jnp.min(jnp.where(mv == cur, iota, L), axis=1, keepdims=True)
        cols.append(idx_t)
        mv = jnp.where(iota == idx_t, _NEG_INF, mv)
    if u < _UPAD:
        cols.append(jnp.full((_N_HEADS, _UPAD - u), L, jnp.int32))
    inds = jnp.concatenate(cols, axis=1)
    o_ref[...] = jnp.concatenate(
        [inds, jnp.zeros((16 - _N_HEADS, _UPAD), jnp.int32)], axis=0)


def _attn_body(q_ref, k_ref, v_ref, i_ref, o_ref, *, L):
    out_cols = []
    row_iota = jax.lax.broadcasted_iota(jnp.int32, (L, _UPAD), 0)
    for h in range(_N_HEADS):
        sl = slice(h * _DH, (h + 1) * _DH)
        q_h = q_ref[:, sl]
        k_h = k_ref[:, sl]
        v_h = v_ref[:, sl]
        idx = i_ref[h:h + 1, :]
        oht = (row_iota == idx).astype(jnp.float32)
        q_red = _dot(oht, q_h, 0, 0)
        scores = _dot(q_red, k_h, 1, 1) * (1.0 / np.sqrt(_DH))
        smax = jnp.max(scores, axis=1, keepdims=True)
        e = jnp.exp(scores - smax)
        p = e / jnp.sum(e, axis=1, keepdims=True)
        upd = _dot(p, v_h, 1, 0)
        ctx = _dot(oht, upd, 1, 0)
        sel = jnp.sum(oht, axis=1, keepdims=True)
        vm = jnp.sum(v_h, axis=0, keepdims=True) * (1.0 / L)
        out_cols.append(ctx + (1.0 - sel) * vm)
    o_ref[...] = jnp.concatenate(out_cols, axis=1)


def _post_body(h_ref, ctx_ref, wo_ref, bo_ref, g1_ref, b1_ref,
               w1_ref, c1_ref, w2_ref, c2_ref, g2_ref, b2_ref, *out_refs):
    x = h_ref[...] + _dot(ctx_ref[...], wo_ref[...], 1, 0) + bo_ref[...]
    x = _ln(x, g1_ref[...], b1_ref[...])
    y = jnp.maximum(_dot(x, w1_ref[...], 1, 0) + c1_ref[...], 0.0)
    y = _dot(y, w2_ref[...], 1, 0) + c2_ref[...]
    out = _ln(x + y, g2_ref[...], b2_ref[...])
    if len(out_refs) == 1:
        out_refs[0][...] = out
    else:
        BL = out.shape[0]
        r2 = 2 * jax.lax.broadcasted_iota(jnp.int32, (BL // 2, BL), 0)
        cj = jax.lax.broadcasted_iota(jnp.int32, (BL // 2, BL), 1)
        out_refs[0][...] = _dot((cj == r2).astype(jnp.float32), out, 1, 0)
        out_refs[1][...] = _dot((cj == r2 + 1).astype(jnp.float32), out, 1, 0)


def _distil_body(he_ref, ho_ref, wt_ref, b_ref, bnm_ref, bnv_ref,
                 bng_ref, bnb_ref, o_ref):
    he = he_ref[...]
    ho = ho_ref[...]

    def bn_elu(z):
        z = z + b_ref[...]
        z = (z - bnm_ref[...]) / jnp.sqrt(bnv_ref[...] + _BN_EPS) \
            * bng_ref[...] + bnb_ref[...]
        return jnp.where(z > 0.0, z, jnp.exp(z) - 1.0)

    w0, w1, w2 = wt_ref[0], wt_ref[1], wt_ref[2]
    e_even = bn_elu(_dot(_shift_down(ho), w0, 1, 0) + _dot(he, w1, 1, 0)
                    + _dot(ho, w2, 1, 0))
    e_odd = bn_elu(_dot(he, w0, 1, 0) + _dot(ho, w1, 1, 0)
                   + _dot(_shift_up(he), w2, 1, 0))
    prev_odd = jnp.concatenate(
        [jnp.full_like(e_odd[:1], _NEG_INF), e_odd[:-1]], axis=0)
    o_ref[...] = jnp.maximum(jnp.maximum(prev_odd, e_even), e_odd)


def _final_body(h_ref, g_ref, b_ref, o_ref, *, L):
    x = _ln(h_ref[...], g_ref[...], b_ref[...])
    o_ref[...] = jnp.sum(x, axis=0, keepdims=True) * (1.0 / L)


def _full(a):
    return pl.BlockSpec(a.shape, lambda: tuple(0 for _ in a.shape))


def _single(body, args, out_shape):
    return pl.pallas_call(
        body,
        in_specs=[_full(a) for a in args],
        out_specs=pl.BlockSpec(out_shape.shape,
                               lambda: tuple(0 for _ in out_shape.shape)),
        out_shape=out_shape,
    )(*args)


def _embed_call(x, wt, pe):
    L = x.shape[0]
    return _single(_embed_body, (x, wt, pe),
                   jax.ShapeDtypeStruct((L, _D_MODEL), jnp.float32))


def _qkv_call(h, p):
    L = h.shape[0]
    BL = 512
    row = lambda n: pl.BlockSpec((1, n), lambda i: (0, 0))
    mat = lambda a: pl.BlockSpec(a.shape, lambda i: (0, 0))
    blk = pl.BlockSpec((BL, _D_MODEL), lambda i: (i, 0))
    return pl.pallas_call(
        _qkv_body,
        grid=(max(L // BL, 1),),
        in_specs=[blk, mat(p['Wq']), mat(p['Wk']), mat(p['Wv']),
                  row(_D_MODEL), row(_D_MODEL), row(_D_MODEL)],
        out_specs=[blk, blk, blk],
        out_shape=[jax.ShapeDtypeStruct((L, _D_MODEL), jnp.float32)] * 3,
    )(h, p['Wq'], p['Wk'], p['Wv'], p['bq'].reshape(1, -1),
      p['bk'].reshape(1, -1), p['bv'].reshape(1, -1))


def _mtopk_call(q, k, cntT, amaskT, u):
    L = q.shape[0]
    body = functools.partial(_mtopk_body, L=L, u=u)
    return _single(body, (q, k, cntT, amaskT),
                   jax.ShapeDtypeStruct((16, _UPAD), jnp.int32))


def _attn_call(q, k, v, inds):
    L = q.shape[0]
    body = functools.partial(_attn_body, L=L)
    return _single(body, (q, k, v, inds),
                   jax.ShapeDtypeStruct((L, _D_MODEL), jnp.float32))


def _post_call(h, ctx, p, split):
    L = h.shape[0]
    BL = 512
    mat = lambda a: pl.BlockSpec(a.shape, lambda i: (0, 0))
    row = lambda n: pl.BlockSpec((1, n), lambda i: (0, 0))
    blk = pl.BlockSpec((BL, _D_MODEL), lambda i: (i, 0))
    if split:
        half = pl.BlockSpec((BL // 2, _D_MODEL), lambda i: (i, 0))
        out_specs = [half, half]
        out_shape = [jax.ShapeDtypeStruct((L // 2, _D_MODEL), jnp.float32)] * 2
    else:
        out_specs = [blk]
        out_shape = [jax.ShapeDtypeStruct((L, _D_MODEL), jnp.float32)]
    return pl.pallas_call(
        _post_body,
        grid=(L // BL,),
        in_specs=[blk, blk, mat(p['Wo']), row(_D_MODEL), row(_D_MODEL),
                  row(_D_MODEL), mat(p['W1']), row(_D_FF), mat(p['W2']),
                  row(_D_MODEL), row(_D_MODEL), row(_D_MODEL)],
        out_specs=out_specs,
        out_shape=out_shape,
    )(h, ctx, p['Wo'], p['bo'].reshape(1, -1), p['g1'].reshape(1, -1),
      p['b1'].reshape(1, -1), p['W1'], p['c1'].reshape(1, -1), p['W2'],
      p['c2'].reshape(1, -1), p['g2'].reshape(1, -1), p['b2'].reshape(1, -1))


def _distil_call(he, ho, p):
    Lh = he.shape[0]
    wt = jnp.transpose(p['w'], (2, 1, 0))
    args = (he, ho, wt, p['b'].reshape(1, -1),
            p['bn_m'].reshape(1, -1), p['bn_v'].reshape(1, -1),
            p['bn_g'].reshape(1, -1), p['bn_b'].reshape(1, -1))
    return _single(_distil_body, args,
                   jax.ShapeDtypeStruct((Lh, _D_MODEL), jnp.float32))


def _final_call(h, g, b):
    body = functools.partial(_final_body, L=h.shape[0])
    return _single(body, (h, g.reshape(1, -1), b.reshape(1, -1)),
                   jax.ShapeDtypeStruct((1, _D_MODEL), jnp.float32))


def kernel(x_enc, params):
    x = x_enc[0] + 1e-10
    L = x.shape[0]

    wt_tok = jnp.transpose(params['tok_w'], (2, 1, 0))
    pe = jnp.asarray(_pe_table(L, _D_MODEL))
    h = _embed_call(x, wt_tok, pe)

    for i in range(_E_LAYERS):
        L = h.shape[0]
        p = params['layers'][i]
        q, k, v = _qkv_call(h, p)
        cntT, amaskT = _sample_consts(i, L)
        inds = _mtopk_call(q, k, jnp.asarray(cntT), jnp.asarray(amaskT),
                           _u_of(L))
        ctx = _attn_call(q, k, v, inds)
        if i < _E_LAYERS - 1:
            he, ho = _post_call(h, ctx, p, True)
            h = _distil_call(he, ho, params['convs'][i])
        else:
            (h,) = _post_call(h, ctx, p, False)

    return _final_call(h, params['norm_g'], params['norm_b'])
